# Initial kernel scaffold; baseline (speedup 1.0000x reference)
import jax
import jax.numpy as jnp
from jax.experimental import pallas as pl


def kernel(x_0e, x_1o, y_0e, y_1o, w_ss, w_vv, w_sv, w_vs, b):
    raise NotImplementedError("write your pallas kernel here")


# speedup vs baseline: 1.8534x; 1.8534x over previous
import numpy as np
import jax
import jax.numpy as jnp
from jax.experimental import pallas as pl
from jax.experimental.pallas import tpu as pltpu

N_ROWS = 100000
MUL = 128
INV_SQRT3_ = 0.5773502691896258
BLOCK = 1000

_T = np.zeros((4, 768), np.float32)
_T[0, :384] = 1.0
_T[1:4, 384:] = np.tile(np.eye(3, dtype=np.float32), (1, 128))
_T_BF16 = jnp.asarray(_T, dtype=jnp.bfloat16)


def _body(x0_ref, x1_ref, m_ref, t_ref, w0_ref, wsv_ref, wvs_ref, b_ref, o_ref):
    bf16 = jnp.bfloat16
    f32 = jnp.float32
    yy = jnp.dot(m_ref[...].astype(bf16), t_ref[...], preferred_element_type=f32)
    y0_384 = yy[:, :384]
    y_tile = yy[:, 384:]

    x0 = x0_ref[...]
    x1 = x1_ref[...]

    seg_ss = x0 * y0_384[:, :128]
    p = x1 * y_tile
    l0 = jnp.concatenate([seg_ss, p], axis=1).astype(bf16)
    out0 = jnp.dot(l0, w0_ref[...], preferred_element_type=f32) + b_ref[...]

    sv = jnp.dot(x0.astype(bf16), wsv_ref[...], preferred_element_type=f32)
    vs = jnp.dot(x1.astype(bf16), wvs_ref[...], preferred_element_type=f32)
    o_ref[:, :128] = out0
    o_ref[:, 128:] = sv * y_tile + vs * y0_384


def kernel(x_0e, x_1o, y_0e, y_1o, w_ss, w_vv, w_sv, w_vs, b):
    n = x_0e.shape[0]
    x1f = x_1o.reshape(n, MUL * 3)
    m = jnp.concatenate([y_0e, y_1o], axis=1)

    bf16 = jnp.bfloat16
    w0 = jnp.concatenate(
        [w_ss, jnp.repeat(w_vv * INV_SQRT3_, 3, axis=0)], axis=0
    ).astype(bf16)
    wsv = jnp.repeat(w_sv, 3, axis=1).astype(bf16)
    wvs = (w_vs[:, None, :, None] * jnp.eye(3, dtype=w_vs.dtype)[None, :, None, :]
           ).reshape(MUL * 3, MUL * 3).astype(bf16)
    b2 = b.reshape(1, MUL)

    grid = n // BLOCK
    row_spec = lambda width: pl.BlockSpec((BLOCK, width), lambda i: (i, 0))
    full_spec = lambda a: pl.BlockSpec(a.shape, lambda i: (0, 0))

    return pl.pallas_call(
        _body,
        grid=(grid,),
        in_specs=[
            row_spec(MUL),
            row_spec(MUL * 3),
            row_spec(4),
            full_spec(_T_BF16),
            full_spec(w0),
            full_spec(wsv),
            full_spec(wvs),
            full_spec(b2),
        ],
        out_specs=row_spec(MUL * 4),
        out_shape=jax.ShapeDtypeStruct((n, MUL * 4), jnp.float32),
        compiler_params=pltpu.CompilerParams(
            dimension_semantics=("parallel",),
            vmem_limit_bytes=50 * 1024 * 1024,
        ),
    )(x_0e, x1f, m, _T_BF16, w0, wsv, wvs, b2)


# speedup vs baseline: 1.8594x; 1.0032x over previous
import numpy as np
import jax
import jax.numpy as jnp
from jax.experimental import pallas as pl
from jax.experimental.pallas import tpu as pltpu

N_ROWS = 100000
MUL = 128
INV_SQRT3_ = 0.5773502691896258
BLOCK = 1000

_T = np.zeros((4, 768), np.float32)
_T[0, :384] = 1.0
_T[1:4, 384:] = np.tile(np.eye(3, dtype=np.float32), (1, 128))


def _body(x0_ref, x1_ref, m_ref, t_ref, w0_ref, wsv_ref, wvs_ref, b_ref, o_ref):
    bf16 = jnp.bfloat16
    f32 = jnp.float32
    yy = jnp.dot(m_ref[...].astype(bf16), t_ref[...], preferred_element_type=f32)
    y0_384 = yy[:, :384]
    y_tile = yy[:, 384:]

    x0 = x0_ref[...]
    x1 = x1_ref[...]

    seg_ss = x0 * y0_384[:, :128]
    p = x1 * y_tile
    l0 = jnp.concatenate([seg_ss, p], axis=1).astype(bf16)
    out0 = jnp.dot(l0, w0_ref[...], preferred_element_type=f32) + b_ref[...]

    sv = jnp.dot(x0.astype(bf16), wsv_ref[...], preferred_element_type=f32)
    vs = jnp.dot(x1.astype(bf16), wvs_ref[...], preferred_element_type=f32)
    o_ref[:, :128] = out0
    o_ref[:, 128:] = sv * y_tile + vs * y0_384


def kernel(x_0e, x_1o, y_0e, y_1o, w_ss, w_vv, w_sv, w_vs, b):
    n = x_0e.shape[0]
    x1f = x_1o.reshape(n, MUL * 3)
    m = jnp.concatenate([y_0e, y_1o], axis=1)

    bf16 = jnp.bfloat16
    t_bf16 = jnp.asarray(_T, dtype=bf16)
    w0 = jnp.concatenate(
        [w_ss, jnp.repeat(w_vv * INV_SQRT3_, 3, axis=0)], axis=0
    ).astype(bf16)
    wsv = jnp.repeat(w_sv, 3, axis=1).astype(bf16)
    wvs = (w_vs[:, None, :, None] * jnp.eye(3, dtype=w_vs.dtype)[None, :, None, :]
           ).reshape(MUL * 3, MUL * 3).astype(bf16)
    b2 = b.reshape(1, MUL)

    grid = n // BLOCK
    row_spec = lambda width: pl.BlockSpec((BLOCK, width), lambda i: (i, 0))
    full_spec = lambda a: pl.BlockSpec(a.shape, lambda i: (0, 0))

    return pl.pallas_call(
        _body,
        grid=(grid,),
        in_specs=[
            row_spec(MUL),
            row_spec(MUL * 3),
            row_spec(4),
            full_spec(t_bf16),
            full_spec(w0),
            full_spec(wsv),
            full_spec(wvs),
            full_spec(b2),
        ],
        out_specs=row_spec(MUL * 4),
        out_shape=jax.ShapeDtypeStruct((n, MUL * 4), jnp.float32),
        compiler_params=pltpu.CompilerParams(
            dimension_semantics=("parallel",),
            vmem_limit_bytes=50 * 1024 * 1024,
        ),
    )(x_0e, x1f, m, t_bf16, w0, wsv, wvs, b2)
